# Initial kernel scaffold; baseline (speedup 1.0000x reference)
import jax
import jax.numpy as jnp
from jax.experimental import pallas as pl


def kernel(x, mem_data, memory, W1, b1, W2, b2, W3, b3):
    raise NotImplementedError("write your pallas kernel here")


# speedup vs baseline: 1.4479x; 1.4479x over previous
import jax
import jax.numpy as jnp
from jax.experimental import pallas as pl
from jax.experimental.pallas import tpu as pltpu

_IN_DIM = 256
_OUT_DIM = 512
_MEM_LEN = 100000
_BETA = 1.0
_BLK_A = 2000
_BLK_B = 2000


def _pass_a_body(in_ref, out_ref, sum_ref, sumsq_ref, acc_s, acc_q):
    i = pl.program_id(0)
    blk = in_ref[...]
    out_ref[...] = blk
    s = jnp.sum(blk, axis=0, keepdims=True)
    q = jnp.sum(blk * blk, axis=0, keepdims=True)

    @pl.when(i == 0)
    def _():
        acc_s[...] = s
        acc_q[...] = q

    @pl.when(i > 0)
    def _():
        acc_s[...] = acc_s[...] + s
        acc_q[...] = acc_q[...] + q

    @pl.when(i == pl.num_programs(0) - 1)
    def _():
        sum_ref[...] = acc_s[...]
        sumsq_ref[...] = acc_q[...]


def _mlp_body(x_ref, s_ref, q_ref, w1, b1, w2, b2, w3, b3, enc_ref):
    n = jnp.float32(_MEM_LEN)
    s = s_ref[...]
    q = q_ref[...]
    mean = s / n
    var = (q - s * (s / n)) / (n - 1.0)
    std = jnp.sqrt(var)
    xn = (x_ref[...] - mean) / std
    xn = jnp.where(std == 0.0, 0.0, xn)
    h1 = jnp.maximum(
        jnp.dot(xn, w1[...], preferred_element_type=jnp.float32) + b1[...], 0.0)
    h2 = jnp.maximum(
        jnp.dot(h1, w2[...], preferred_element_type=jnp.float32) + b2[...], 0.0)
    enc_ref[...] = jnp.tanh(
        jnp.dot(h2, w3[...], preferred_element_type=jnp.float32) + b3[...])


def _pass_b_body(mem_ref, enc_ref, out_ref, loss_ref, min_s):
    i = pl.program_id(0)
    blk = mem_ref[...]
    out_ref[...] = blk
    m = jnp.min(jnp.sum(jnp.abs(blk - enc_ref[...]), axis=1))

    @pl.when(i == 0)
    def _():
        min_s[0] = m

    @pl.when(i > 0)
    def _():
        min_s[0] = jnp.minimum(min_s[0], m)

    @pl.when(i == pl.num_programs(0) - 1)
    def _():
        loss_ref[0, 0] = min_s[0]


def _fixup_body(mem_in, md_in, loss_ref, enc_ref, x_ref, mem_out, md_out):
    mem_out[...] = mem_in[...]
    md_out[...] = md_in[...]

    @pl.when(loss_ref[0, 0] <= _BETA)
    def _():
        mem_out[0:1, :] = enc_ref[...]
        md_out[0:1, :] = x_ref[...]


def kernel(x, mem_data, memory, W1, b1, W2, b2, W3, b3):
    f32 = jnp.float32
    W1p = jnp.pad(W1, ((0, 0), (0, 12)))
    b1p = jnp.pad(b1, (0, 12)).reshape(1, 512)
    W2p = jnp.pad(W2, ((0, 12), (0, 24)))
    b2p = jnp.pad(b2, (0, 24)).reshape(1, 1024)
    W3p = jnp.pad(W3, ((0, 24), (0, 0)))
    b3p = b3.reshape(1, 512)

    na = _MEM_LEN // _BLK_A
    new_mem_data, col_sum, col_sumsq = pl.pallas_call(
        _pass_a_body,
        grid=(na,),
        in_specs=[pl.BlockSpec((_BLK_A, _IN_DIM), lambda i: (i, 0))],
        out_specs=[
            pl.BlockSpec((_BLK_A, _IN_DIM), lambda i: (i, 0)),
            pl.BlockSpec((1, _IN_DIM), lambda i: (0, 0)),
            pl.BlockSpec((1, _IN_DIM), lambda i: (0, 0)),
        ],
        out_shape=[
            jax.ShapeDtypeStruct((_MEM_LEN, _IN_DIM), f32),
            jax.ShapeDtypeStruct((1, _IN_DIM), f32),
            jax.ShapeDtypeStruct((1, _IN_DIM), f32),
        ],
        scratch_shapes=[
            pltpu.VMEM((1, _IN_DIM), f32),
            pltpu.VMEM((1, _IN_DIM), f32),
        ],
    )(mem_data)

    enc = pl.pallas_call(
        _mlp_body,
        out_shape=jax.ShapeDtypeStruct((1, _OUT_DIM), f32),
    )(x, col_sum, col_sumsq, W1p, b1p, W2p, b2p, W3p, b3p)

    nb = _MEM_LEN // _BLK_B
    new_memory, loss11 = pl.pallas_call(
        _pass_b_body,
        grid=(nb,),
        in_specs=[
            pl.BlockSpec((_BLK_B, _OUT_DIM), lambda i: (i, 0)),
            pl.BlockSpec((1, _OUT_DIM), lambda i: (0, 0)),
        ],
        out_specs=[
            pl.BlockSpec((_BLK_B, _OUT_DIM), lambda i: (i, 0)),
            pl.BlockSpec(memory_space=pltpu.SMEM),
        ],
        out_shape=[
            jax.ShapeDtypeStruct((_MEM_LEN, _OUT_DIM), f32),
            jax.ShapeDtypeStruct((1, 1), f32),
        ],
        scratch_shapes=[pltpu.SMEM((1,), f32)],
    )(memory, enc)

    new_memory, new_mem_data = pl.pallas_call(
        _fixup_body,
        grid=(1,),
        in_specs=[
            pl.BlockSpec((8, _OUT_DIM), lambda i: (0, 0)),
            pl.BlockSpec((8, _IN_DIM), lambda i: (0, 0)),
            pl.BlockSpec(memory_space=pltpu.SMEM),
            pl.BlockSpec((1, _OUT_DIM), lambda i: (0, 0)),
            pl.BlockSpec((1, _IN_DIM), lambda i: (0, 0)),
        ],
        out_specs=[
            pl.BlockSpec((8, _OUT_DIM), lambda i: (0, 0)),
            pl.BlockSpec((8, _IN_DIM), lambda i: (0, 0)),
        ],
        out_shape=[
            jax.ShapeDtypeStruct((_MEM_LEN, _OUT_DIM), f32),
            jax.ShapeDtypeStruct((_MEM_LEN, _IN_DIM), f32),
        ],
        input_output_aliases={0: 0, 1: 1},
    )(new_memory, new_mem_data, loss11, enc, x)

    return loss11[0, 0], new_memory, new_mem_data
